# baseline (speedup 1.0000x reference)
import functools

import jax
import jax.numpy as jnp
from jax import lax
from jax.experimental import pallas as pl
from jax.experimental.pallas import tpu as pltpu
from jax.experimental.pallas import tpu_sc as plsc

_B, _F, _H, _W, _C = 1, 64, 32, 32, 768
_T = _H * _W
_EPS = 1e-12

_F_SC = 24
_F_TC = _F - _F_SC


_NC, _NS, _L = 2, 16, 16
_NW = _NC * _NS
_TPW = _T // _NW
_SLAB = _TPW * _C
_NCHUNK = _SLAB // _L
_NBUF = 4


def _sc_sum_body(g_hbm, out_hbm, b0, b1, b2, b3, acc, s0, s1, s2, s3):
    bufs = (b0, b1, b2, b3)
    sems = (s0, s1, s2, s3)
    wid = lax.axis_index("s") * _NC + lax.axis_index("c")

    zeros = jnp.zeros((_L,), jnp.float32)

    def zrow(r, c):
        @plsc.parallel_loop(0, _C, step=_L, unroll=8)
        def _z(i):
            acc[r, pl.ds(i, _L)] = zeros
        return c

    lax.fori_loop(0, _W, zrow, 0)

    def fire(f, b):
        pltpu.async_copy(g_hbm.at[f, wid], bufs[b], sems[b])

    def wait(b):
        pltpu.make_async_copy(g_hbm.at[0, 0], bufs[b], sems[b]).wait()

    for b in range(_NBUF):
        fire(b, b)

    def round_body(r, c):
        for b in range(_NBUF):
            wait(b)

            def arow(rr, cc):
                @plsc.parallel_loop(0, _C, step=_L, unroll=8)
                def _a(i):
                    sl = pl.ds(i, _L)
                    plsc.addupdate(acc.at[rr, sl], bufs[b][rr, sl])
                return cc

            lax.fori_loop(0, _W, arow, 0)

            nxt = (r + 1) * _NBUF + b

            @pl.when(nxt < _F_SC)
            def _refire():
                fire(nxt, b)

        return c

    lax.fori_loop(0, _F_SC // _NBUF, round_body, 0)
    pltpu.sync_copy(acc, out_hbm.at[wid])


def _sc_sum(g4d):
    mesh = plsc.VectorSubcoreMesh(core_axis_name="c", subcore_axis_name="s")
    return pl.kernel(
        _sc_sum_body,
        out_type=jax.ShapeDtypeStruct((_H, _W, _C), jnp.float32),
        mesh=mesh,
        compiler_params=pltpu.CompilerParams(use_tc_tiling_on_sc=True),
        scratch_types=(
            [pltpu.VMEM((_W, _C), jnp.float32)] * _NBUF
            + [pltpu.VMEM((_W, _C), jnp.float32)]
            + [pltpu.SemaphoreType.DMA] * _NBUF
        ),
    )(g4d)


_FB = 4


def _tc_sum_body(g_ref, out_ref, acc_ref):
    f = pl.program_id(0)
    part = g_ref[0]
    for i in range(1, _FB):
        part = part + g_ref[i]

    @pl.when(f == 0)
    def _init():
        acc_ref[...] = part

    @pl.when(f > 0)
    def _accum():
        acc_ref[...] += part

    @pl.when(f == (_F_TC // _FB) - 1)
    def _finish():
        out_ref[...] = acc_ref[...]


def _combine_body(s_ref, row_ref, col_ref, tte_ref, w_ref, b_ref, out_ref):
    x = s_ref[...] * (1.0 / _F)
    x = x + row_ref[...][:, None, :] + col_ref[...][None, :, :]
    x = x + tte_ref[...][None, :, :]
    mu = jnp.mean(x, axis=-1, keepdims=True)
    var = jnp.mean(jnp.square(x - mu), axis=-1, keepdims=True)
    y = (x - mu) * jax.lax.rsqrt(var + _EPS)
    out_ref[...] = y * w_ref[...][None, :, :] + b_ref[...][None, :, :]


def _combine2_body(s_ref, t_ref, row_ref, col_ref, tte_ref, w_ref, b_ref,
                   out_ref):
    x = (s_ref[...] + t_ref[...]) * (1.0 / _F)
    x = x + row_ref[...][:, None, :] + col_ref[...][None, :, :]
    x = x + tte_ref[...][None, :, :]
    mu = jnp.mean(x, axis=-1, keepdims=True)
    var = jnp.mean(jnp.square(x - mu), axis=-1, keepdims=True)
    y = (x - mu) * jax.lax.rsqrt(var + _EPS)
    out_ref[...] = y * w_ref[...][None, :, :] + b_ref[...][None, :, :]


_WHOLE = pl.BlockSpec((_H, _W, _C), lambda: (0, 0, 0))
_EMB_SPECS = [
    pl.BlockSpec((_H, _C), lambda: (0, 0)),
    pl.BlockSpec((_W, _C), lambda: (0, 0)),
    pl.BlockSpec((1, _C), lambda: (0, 0)),
    pl.BlockSpec((1, _C), lambda: (0, 0)),
    pl.BlockSpec((1, _C), lambda: (0, 0)),
]


def kernel(grid, row_emb, col_emb, token_type_emb, ln_weight, ln_bias):
    g = grid.reshape(_F, _H, _W, _C)
    w2 = ln_weight.reshape(1, _C)
    b2 = ln_bias.reshape(1, _C)

    sc_part = _sc_sum(g[_F_TC:])

    if _F_TC:
        tc_part = pl.pallas_call(
            _tc_sum_body,
            grid=(_F_TC // _FB,),
            in_specs=[pl.BlockSpec((_FB, _H, _W, _C), lambda f: (f, 0, 0, 0))],
            out_specs=pl.BlockSpec((_H, _W, _C), lambda f: (0, 0, 0)),
            out_shape=jax.ShapeDtypeStruct((_H, _W, _C), jnp.float32),
            scratch_shapes=[pltpu.VMEM((_H, _W, _C), jnp.float32)],
        )(g[:_F_TC])
        out = pl.pallas_call(
            _combine2_body,
            in_specs=[_WHOLE, _WHOLE] + _EMB_SPECS,
            out_specs=_WHOLE,
            out_shape=jax.ShapeDtypeStruct((_H, _W, _C), jnp.float32),
        )(sc_part, tc_part, row_emb, col_emb, token_type_emb, w2, b2)
    else:
        out = pl.pallas_call(
            _combine_body,
            in_specs=[_WHOLE] + _EMB_SPECS,
            out_specs=_WHOLE,
            out_shape=jax.ShapeDtypeStruct((_H, _W, _C), jnp.float32),
        )(sc_part, row_emb, col_emb, token_type_emb, w2, b2)
    return out.reshape(_B, _T, _C)


# speedup vs baseline: 2.4458x; 2.4458x over previous
import functools

import jax
import jax.numpy as jnp
from jax import lax
from jax.experimental import pallas as pl
from jax.experimental.pallas import tpu as pltpu
from jax.experimental.pallas import tpu_sc as plsc

_B, _F, _H, _W, _C = 1, 64, 32, 32, 768
_T = _H * _W
_EPS = 1e-12

_F_SC = 24
_F_TC = _F - _F_SC


_NC, _NS, _L = 2, 16, 16
_NW = _NC * _NS
_TPW = _T // _NW
_SLAB = _TPW * _C
_NCHUNK = _SLAB // _L
_NBUF = 4


def _sc_sum_body(g_hbm, out_hbm, b0, b1, b2, b3, acc, s0, s1, s2, s3):
    bufs = (b0, b1, b2, b3)
    sems = (s0, s1, s2, s3)
    wid = lax.axis_index("s") * _NC + lax.axis_index("c")

    zeros = jnp.zeros((_L,), jnp.float32)

    def zrow(r, c):
        @plsc.parallel_loop(0, _C, step=_L, unroll=8)
        def _z(i):
            acc[r, pl.ds(i, _L)] = zeros
        return c

    lax.fori_loop(0, _W, zrow, 0)

    def fire(f, b):
        pltpu.async_copy(g_hbm.at[_F_TC + f, wid], bufs[b], sems[b])

    def wait(b):
        pltpu.make_async_copy(g_hbm.at[0, 0], bufs[b], sems[b]).wait()

    for b in range(_NBUF):
        fire(b, b)

    def round_body(r, c):
        for b in range(_NBUF):
            wait(b)

            def arow(rr, cc):
                @plsc.parallel_loop(0, _C, step=_L, unroll=8)
                def _a(i):
                    sl = pl.ds(i, _L)
                    plsc.addupdate(acc.at[rr, sl], bufs[b][rr, sl])
                return cc

            lax.fori_loop(0, _W, arow, 0)

            nxt = (r + 1) * _NBUF + b

            @pl.when(nxt < _F_SC)
            def _refire():
                fire(nxt, b)

        return c

    lax.fori_loop(0, _F_SC // _NBUF, round_body, 0)
    pltpu.sync_copy(acc, out_hbm.at[wid])


def _sc_sum(g4d):
    mesh = plsc.VectorSubcoreMesh(core_axis_name="c", subcore_axis_name="s")
    return pl.kernel(
        _sc_sum_body,
        out_type=jax.ShapeDtypeStruct((_H, _W, _C), jnp.float32),
        mesh=mesh,
        compiler_params=pltpu.CompilerParams(use_tc_tiling_on_sc=True),
        scratch_types=(
            [pltpu.VMEM((_W, _C), jnp.float32)] * _NBUF
            + [pltpu.VMEM((_W, _C), jnp.float32)]
            + [pltpu.SemaphoreType.DMA] * _NBUF
        ),
    )(g4d)


_FB = 4


def _tc_sum_body(g_ref, out_ref, acc_ref):
    f = pl.program_id(0)
    part = g_ref[0]
    for i in range(1, _FB):
        part = part + g_ref[i]

    @pl.when(f == 0)
    def _init():
        acc_ref[...] = part

    @pl.when(f > 0)
    def _accum():
        acc_ref[...] += part

    @pl.when(f == (_F_TC // _FB) - 1)
    def _finish():
        out_ref[...] = acc_ref[...]


def _combine_body(s_ref, row_ref, col_ref, tte_ref, w_ref, b_ref, out_ref):
    x = s_ref[...] * (1.0 / _F)
    x = x + row_ref[...][:, None, :] + col_ref[...][None, :, :]
    x = x + tte_ref[...][None, :, :]
    mu = jnp.mean(x, axis=-1, keepdims=True)
    var = jnp.mean(jnp.square(x - mu), axis=-1, keepdims=True)
    y = (x - mu) * jax.lax.rsqrt(var + _EPS)
    out_ref[...] = y * w_ref[...][None, :, :] + b_ref[...][None, :, :]


def _combine2_body(s_ref, t_ref, row_ref, col_ref, tte_ref, w_ref, b_ref,
                   out_ref):
    x = (s_ref[...] + t_ref[...]) * (1.0 / _F)
    x = x + row_ref[...][:, None, :] + col_ref[...][None, :, :]
    x = x + tte_ref[...][None, :, :]
    mu = jnp.mean(x, axis=-1, keepdims=True)
    var = jnp.mean(jnp.square(x - mu), axis=-1, keepdims=True)
    y = (x - mu) * jax.lax.rsqrt(var + _EPS)
    out_ref[...] = y * w_ref[...][None, :, :] + b_ref[...][None, :, :]


_WHOLE = pl.BlockSpec((_H, _W, _C), lambda: (0, 0, 0))
_EMB_SPECS = [
    pl.BlockSpec((_H, _C), lambda: (0, 0)),
    pl.BlockSpec((_W, _C), lambda: (0, 0)),
    pl.BlockSpec((1, _C), lambda: (0, 0)),
    pl.BlockSpec((1, _C), lambda: (0, 0)),
    pl.BlockSpec((1, _C), lambda: (0, 0)),
]


def kernel(grid, row_emb, col_emb, token_type_emb, ln_weight, ln_bias):
    g = grid.reshape(_F, _H, _W, _C)
    w2 = ln_weight.reshape(1, _C)
    b2 = ln_bias.reshape(1, _C)

    sc_part = _sc_sum(g)

    if _F_TC:
        tc_part = pl.pallas_call(
            _tc_sum_body,
            grid=(_F_TC // _FB,),
            in_specs=[pl.BlockSpec((_FB, _H, _W, _C), lambda f: (f, 0, 0, 0))],
            out_specs=pl.BlockSpec((_H, _W, _C), lambda f: (0, 0, 0)),
            out_shape=jax.ShapeDtypeStruct((_H, _W, _C), jnp.float32),
            scratch_shapes=[pltpu.VMEM((_H, _W, _C), jnp.float32)],
        )(g)
        out = pl.pallas_call(
            _combine2_body,
            in_specs=[_WHOLE, _WHOLE] + _EMB_SPECS,
            out_specs=_WHOLE,
            out_shape=jax.ShapeDtypeStruct((_H, _W, _C), jnp.float32),
        )(sc_part, tc_part, row_emb, col_emb, token_type_emb, w2, b2)
    else:
        out = pl.pallas_call(
            _combine_body,
            in_specs=[_WHOLE] + _EMB_SPECS,
            out_specs=_WHOLE,
            out_shape=jax.ShapeDtypeStruct((_H, _W, _C), jnp.float32),
        )(sc_part, row_emb, col_emb, token_type_emb, w2, b2)
    return out.reshape(_B, _T, _C)


# speedup vs baseline: 2.6321x; 1.0762x over previous
import jax
import jax.numpy as jnp
from jax.experimental import pallas as pl
from jax.experimental.pallas import tpu as pltpu

_B, _F, _H, _W, _C = 1, 64, 32, 32, 768
_T = _H * _W
_EPS = 1e-12

_FB = 4
_HB = 8
_NH = _H // _HB
_NFS = _F // _FB


def _body(g_ref, row_ref, col_ref, tte_ref, w_ref, b_ref, out_ref, acc_ref):
    fs = pl.program_id(1)
    hb = pl.program_id(0)
    part = g_ref[0]
    for i in range(1, _FB):
        part = part + g_ref[i]

    @pl.when(fs == 0)
    def _init():
        acc_ref[...] = part

    @pl.when(fs > 0)
    def _accum():
        acc_ref[...] += part

    @pl.when(fs == _NFS - 1)
    def _finish():
        x = acc_ref[...] * (1.0 / _F)
        row = row_ref[pl.ds(hb * _HB, _HB)]
        x = x + row[:, None, :] + col_ref[...][None, :, :]
        x = x + tte_ref[...][None, :, :]
        mu = jnp.mean(x, axis=-1, keepdims=True)
        var = jnp.mean(jnp.square(x - mu), axis=-1, keepdims=True)
        y = (x - mu) * jax.lax.rsqrt(var + _EPS)
        out_ref[...] = y * w_ref[...][None, :, :] + b_ref[...][None, :, :]


def kernel(grid, row_emb, col_emb, token_type_emb, ln_weight, ln_bias):
    g = grid.reshape(_F, _H, _W, _C)
    w2 = ln_weight.reshape(1, _C)
    b2 = ln_bias.reshape(1, _C)
    out = pl.pallas_call(
        _body,
        grid=(_NH, _NFS),
        in_specs=[
            pl.BlockSpec((_FB, _HB, _W, _C), lambda hb, fs: (fs, hb, 0, 0)),
            pl.BlockSpec((_H, _C), lambda hb, fs: (0, 0)),
            pl.BlockSpec((_W, _C), lambda hb, fs: (0, 0)),
            pl.BlockSpec((1, _C), lambda hb, fs: (0, 0)),
            pl.BlockSpec((1, _C), lambda hb, fs: (0, 0)),
            pl.BlockSpec((1, _C), lambda hb, fs: (0, 0)),
        ],
        out_specs=pl.BlockSpec((_HB, _W, _C), lambda hb, fs: (hb, 0, 0)),
        out_shape=jax.ShapeDtypeStruct((_H, _W, _C), jnp.float32),
        scratch_shapes=[pltpu.VMEM((_HB, _W, _C), jnp.float32)],
    )(g, row_emb, col_emb, token_type_emb, w2, b2)
    return out.reshape(_B, _T, _C)


# speedup vs baseline: 3.2286x; 1.2266x over previous
import jax
import jax.numpy as jnp
from jax.experimental import pallas as pl
from jax.experimental.pallas import tpu as pltpu

_B, _F, _H, _W, _C = 1, 64, 32, 32, 768
_T = _H * _W
_EPS = 1e-12

_FB = 8
_HB = 16
_NH = _H // _HB
_NFS = _F // _FB


def _body(g_ref, row_ref, col_ref, tte_ref, w_ref, b_ref, out_ref, acc_ref):
    fs = pl.program_id(1)
    hb = pl.program_id(0)
    part = g_ref[0]
    for i in range(1, _FB):
        part = part + g_ref[i]

    @pl.when(fs == 0)
    def _init():
        acc_ref[...] = part

    @pl.when(fs > 0)
    def _accum():
        acc_ref[...] += part

    @pl.when(fs == _NFS - 1)
    def _finish():
        x = acc_ref[...] * (1.0 / _F)
        row = row_ref[pl.ds(hb * _HB, _HB)]
        x = x + row[:, None, :] + col_ref[...][None, :, :]
        x = x + tte_ref[...][None, :, :]
        mu = jnp.mean(x, axis=-1, keepdims=True)
        var = jnp.mean(jnp.square(x - mu), axis=-1, keepdims=True)
        y = (x - mu) * jax.lax.rsqrt(var + _EPS)
        out_ref[...] = y * w_ref[...][None, :, :] + b_ref[...][None, :, :]


def kernel(grid, row_emb, col_emb, token_type_emb, ln_weight, ln_bias):
    g = grid.reshape(_F, _H, _W, _C)
    w2 = ln_weight.reshape(1, _C)
    b2 = ln_bias.reshape(1, _C)
    out = pl.pallas_call(
        _body,
        grid=(_NH, _NFS),
        in_specs=[
            pl.BlockSpec((_FB, _HB, _W, _C), lambda hb, fs: (fs, hb, 0, 0)),
            pl.BlockSpec((_H, _C), lambda hb, fs: (0, 0)),
            pl.BlockSpec((_W, _C), lambda hb, fs: (0, 0)),
            pl.BlockSpec((1, _C), lambda hb, fs: (0, 0)),
            pl.BlockSpec((1, _C), lambda hb, fs: (0, 0)),
            pl.BlockSpec((1, _C), lambda hb, fs: (0, 0)),
        ],
        out_specs=pl.BlockSpec((_HB, _W, _C), lambda hb, fs: (hb, 0, 0)),
        out_shape=jax.ShapeDtypeStruct((_H, _W, _C), jnp.float32),
        scratch_shapes=[pltpu.VMEM((_HB, _W, _C), jnp.float32)],
    )(g, row_emb, col_emb, token_type_emb, w2, b2)
    return out.reshape(_B, _T, _C)
